# baseline (speedup 1.0000x reference)
import jax
import jax.numpy as jnp
from jax.experimental import pallas as pl
from jax.experimental.pallas import tpu as pltpu

_B, _D, _P = 64, 96, 1024
_CB = 8
_NCH = _B // _CB
_NBUF = 6


def _add_body(x_hbm, p_ref, o_hbm, ibuf, obuf, isems, osems):
    pos = p_ref[...]

    def in_cp(c, s):
        return pltpu.make_async_copy(
            x_hbm.at[pl.ds(c * _CB, _CB)], ibuf.at[s], isems.at[s])

    def out_cp(c, s):
        return pltpu.make_async_copy(
            obuf.at[s], o_hbm.at[pl.ds(c * _CB, _CB)], osems.at[s])

    for s in range(_NBUF):
        in_cp(s, s).start()
    for c in range(_NCH):
        s = c % _NBUF
        in_cp(c, s).wait()
        if c >= _NBUF:
            out_cp(c - _NBUF, s).wait()
        obuf[s] = ibuf[s] + pos
        out_cp(c, s).start()
        if c + _NBUF < _NCH:
            in_cp(c + _NBUF, s).start()
    for c in range(_NCH - _NBUF, _NCH):
        out_cp(c, c % _NBUF).wait()


def kernel(x, pos_table):
    xt = jnp.swapaxes(x, 1, 2)
    pt = jnp.swapaxes(pos_table, 0, 1)
    out = pl.pallas_call(
        _add_body,
        in_specs=[
            pl.BlockSpec(memory_space=pl.ANY),
            pl.BlockSpec(memory_space=pltpu.MemorySpace.VMEM),
        ],
        out_specs=pl.BlockSpec(memory_space=pl.ANY),
        out_shape=jax.ShapeDtypeStruct((_B, _D, _P), jnp.float32),
        scratch_shapes=[
            pltpu.VMEM((_NBUF, _CB, _D, _P), jnp.float32),
            pltpu.VMEM((_NBUF, _CB, _D, _P), jnp.float32),
            pltpu.SemaphoreType.DMA((_NBUF,)),
            pltpu.SemaphoreType.DMA((_NBUF,)),
        ],
    )(xt, pt)
    return jnp.swapaxes(out, 1, 2)
